# Initial kernel scaffold; baseline (speedup 1.0000x reference)
import functools

import jax
import jax.numpy as jnp
from jax import lax
from jax.experimental import pallas as pl
from jax.experimental.pallas import tpu as pltpu
from jax.experimental.pallas import tpu_sc as plsc

N_NODES = 10000
N_EDGES = 320000
D = 128

NW = 32
NPT = 313
NPAD = NW * NPT
ACC_ROWS = 320
DUMMY_ROW = 318
SRC_BITS = 14
CHUNK_F = 8000
LIST_STRIDE = N_EDGES + 1024
CHUNK_G = 128

_mesh = plsc.VectorSubcoreMesh(core_axis_name="c", subcore_axis_name="s")


def _wid():
    return lax.axis_index("s") * 2 + lax.axis_index("c")


@functools.partial(
    pl.kernel,
    out_type=(
        jax.ShapeDtypeStruct((NW, LIST_STRIDE), jnp.int32),
        jax.ShapeDtypeStruct((NW, 16), jnp.int32),
    ),
    mesh=_mesh,
    scratch_types=[
        pltpu.VMEM((CHUNK_F,), jnp.int32),
        pltpu.VMEM((CHUNK_F,), jnp.int32),
        pltpu.VMEM((CHUNK_F + 16,), jnp.int32),
        pltpu.VMEM((16,), jnp.int32),
    ],
)
def _partition_edges(src_hbm, dst_hbm, list_hbm, cnt_hbm, srcb, dstb, outb,
                     cntb):
    wid = _wid()
    lo = wid * NPT
    dummy = jnp.full((16,), DUMMY_ROW << SRC_BITS, jnp.int32)

    def chunk_body(c, g):
        off = c * CHUNK_F
        pltpu.sync_copy(src_hbm.at[pl.ds(off, CHUNK_F)], srcb)
        pltpu.sync_copy(dst_hbm.at[pl.ds(off, CHUNK_F)], dstb)

        def step(i, lc):
            vd = dstb[pl.ds(i * 16, 16)]
            vs = srcb[pl.ds(i * 16, 16)]
            dl = vd - lo
            msk = (dl >= 0) & (dl < NPT)
            w = (dl << SRC_BITS) | vs
            outb[pl.ds(lc, 16)] = dummy
            plsc.store_compressed(outb.at[pl.ds(lc, 16)], w, msk)
            return lc + jnp.sum(msk.astype(jnp.int32))

        lc = lax.fori_loop(0, CHUNK_F // 16, step, 0)
        outb[pl.ds(lc, 16)] = dummy
        lc = (lc + 7) // 8 * 8
        pltpu.sync_copy(outb, list_hbm.at[wid, pl.ds(g, CHUNK_F + 16)])
        return g + lc

    total = lax.fori_loop(0, N_EDGES // CHUNK_F, chunk_body, 0)
    cntb[...] = jnp.full((16,), 1, jnp.int32) * total
    pltpu.sync_copy(cntb, cnt_hbm.at[wid])


@functools.partial(
    pl.kernel,
    out_type=jax.ShapeDtypeStruct((NPAD, D), jnp.float32),
    mesh=_mesh,
    scratch_types=[
        pltpu.VMEM((ACC_ROWS, D), jnp.float32),
        pltpu.VMEM((CHUNK_G,), jnp.int32),
        pltpu.VMEM((CHUNK_G,), jnp.int32),
        pltpu.VMEM((CHUNK_G,), jnp.int32),
        pltpu.VMEM((CHUNK_G, D), jnp.float32),
        pltpu.VMEM((16,), jnp.int32),
        pltpu.SemaphoreType.DMA,
    ],
)
def _gather_segmax(m_hbm, list_hbm, cnt_hbm, agg_hbm, acc, idxb, dstb, wb,
                   rows, cntb, sem):
    wid = _wid()
    zero = jnp.zeros((16,), jnp.float32)

    def zrow(i, _):
        for j in range(D // 16):
            acc[i, pl.ds(j * 16, 16)] = zero
        return 0

    lax.fori_loop(0, ACC_ROWS, zrow, 0)

    pltpu.sync_copy(cnt_hbm.at[wid], cntb)
    k_total = cntb[0]
    nch = (k_total + CHUNK_G - 1) // CHUNK_G

    def chunk(c, _):
        e0 = c * CHUNK_G
        pltpu.sync_copy(list_hbm.at[wid, pl.ds(e0, CHUNK_G)], wb)

        def unpack(i, _):
            w = wb[pl.ds(i * 16, 16)]
            pos = e0 + i * 16 + lax.iota(jnp.int32, 16)
            valid = pos < k_total
            src = w & ((1 << SRC_BITS) - 1)
            dl = lax.shift_right_logical(w, SRC_BITS)
            idxb[pl.ds(i * 16, 16)] = jnp.where(valid, src, 0)
            dstb[pl.ds(i * 16, 16)] = jnp.where(valid, dl, DUMMY_ROW)
            return 0

        lax.fori_loop(0, CHUNK_G // 16, unpack, 0)
        pltpu.async_copy(m_hbm.at[idxb], rows, sem).wait()

        def edge(e, _):
            dloc = dstb[e]
            for j in range(D // 16):
                sl = pl.ds(j * 16, 16)
                acc[dloc, sl] = jnp.maximum(acc[dloc, sl], rows[e, sl])
            return 0

        lax.fori_loop(0, CHUNK_G, edge, 0)
        return 0

    lax.fori_loop(0, nch, chunk, 0)
    pltpu.sync_copy(acc.at[pl.ds(0, NPT)], agg_hbm.at[pl.ds(wid * NPT, NPT)])


_ROWS_BLK = 400
_GRID = N_NODES // _ROWS_BLK


def _dot(a, b):
    return jnp.dot(a, b, preferred_element_type=jnp.float32)


def _tc_stage_a(x_ref, wp, bp, ws, bs, m_ref, s_ref):
    xb = x_ref[...]
    m_ref[...] = jnp.maximum(_dot(xb, wp[...]) + bp[...], 0.0)
    s_ref[...] = _dot(xb, ws[...]) + bs[...]


def _tc_stage_b(s1_ref, agg_ref, wn, bn, wp2, bp2, ws2, bs2, m2_ref, s2_ref):
    h = jnp.maximum(s1_ref[...] + _dot(agg_ref[...], wn[...]) + bn[...], 0.0)
    m2_ref[...] = jnp.maximum(_dot(h, wp2[...]) + bp2[...], 0.0)
    s2_ref[...] = _dot(h, ws2[...]) + bs2[...]


def _tc_stage_c(s2_ref, agg_ref, wn2, bn2, out_ref):
    o = s2_ref[...] + _dot(agg_ref[...], wn2[...]) + bn2[...]
    mx = jnp.max(o, axis=1, keepdims=True)
    e = jnp.exp(o - mx)
    out_ref[...] = (o - mx) - jnp.log(jnp.sum(e, axis=1, keepdims=True))


_row_spec = pl.BlockSpec((_ROWS_BLK, D), lambda i: (i, 0))
_w_spec = pl.BlockSpec((D, D), lambda i: (0, 0))
_b_spec = pl.BlockSpec((1, D), lambda i: (0, 0))
_fdt = jax.ShapeDtypeStruct((N_NODES, D), jnp.float32)


def kernel(x, edge_index, W_pool1, b_pool1, W_self1, b_self1, W_neigh1,
           b_neigh1, W_pool2, b_pool2, W_self2, b_self2, W_neigh2, b_neigh2):
    src = edge_index[0]
    dst = edge_index[1]

    edge_list, edge_cnt = _partition_edges(src, dst)

    m1, s1 = pl.pallas_call(
        _tc_stage_a,
        grid=(_GRID,),
        in_specs=[_row_spec, _w_spec, _b_spec, _w_spec, _b_spec],
        out_specs=[_row_spec, _row_spec],
        out_shape=[_fdt, _fdt],
    )(x, W_pool1, b_pool1.reshape(1, D), W_self1, b_self1.reshape(1, D))

    agg1 = _gather_segmax(m1, edge_list, edge_cnt)[:N_NODES]

    m2, s2 = pl.pallas_call(
        _tc_stage_b,
        grid=(_GRID,),
        in_specs=[_row_spec, _row_spec, _w_spec, _b_spec, _w_spec, _b_spec,
                  _w_spec, _b_spec],
        out_specs=[_row_spec, _row_spec],
        out_shape=[_fdt, _fdt],
    )(s1, agg1, W_neigh1, b_neigh1.reshape(1, D), W_pool2,
      b_pool2.reshape(1, D), W_self2, b_self2.reshape(1, D))

    agg2 = _gather_segmax(m2, edge_list, edge_cnt)[:N_NODES]

    out = pl.pallas_call(
        _tc_stage_c,
        grid=(_GRID,),
        in_specs=[_row_spec, _row_spec, _w_spec, _b_spec],
        out_specs=_row_spec,
        out_shape=_fdt,
    )(s2, agg2, W_neigh2, b_neigh2.reshape(1, D))

    return out


# speedup vs baseline: 2.2954x; 2.2954x over previous
import functools

import jax
import jax.numpy as jnp
from jax import lax
from jax.experimental import pallas as pl
from jax.experimental.pallas import tpu as pltpu
from jax.experimental.pallas import tpu_sc as plsc

N_NODES = 10000
N_EDGES = 320000
D = 128

NW = 32
NPT = 320
NPAD = NW * NPT
ACC_ROWS = 328
DUMMY_ROW = 324
SRC_BITS = 14
CHUNK_F = 8000
LIST_STRIDE = N_EDGES + 1024
CHUNK_G = 128

_mesh = plsc.VectorSubcoreMesh(core_axis_name="c", subcore_axis_name="s",
                               num_cores=2, num_subcores=16)


def _wid():
    return lax.axis_index("s") * 2 + lax.axis_index("c")


@functools.partial(
    pl.kernel,
    out_type=(
        jax.ShapeDtypeStruct((NW * LIST_STRIDE,), jnp.int32),
        jax.ShapeDtypeStruct((NW * 16,), jnp.int32),
    ),
    mesh=_mesh,
    compiler_params=pltpu.CompilerParams(needs_layout_passes=False),
    scratch_types=[
        pltpu.VMEM((CHUNK_F,), jnp.int32),
        pltpu.VMEM((CHUNK_F,), jnp.int32),
        pltpu.VMEM((CHUNK_F + 16,), jnp.int32),
        pltpu.VMEM((16,), jnp.int32),
    ],
)
def _partition_edges(src_hbm, dst_hbm, list_hbm, cnt_hbm, srcb,
                     dstb, outb, cntb):
    wid = _wid()
    lo = wid * NPT
    dummy = jnp.full((16,), DUMMY_ROW << SRC_BITS, jnp.int32)

    def chunk_body(c, g):
        off = pl.multiple_of(c * CHUNK_F, 8)
        pltpu.sync_copy(src_hbm.at[pl.ds(off, CHUNK_F)], srcb)
        pltpu.sync_copy(dst_hbm.at[pl.ds(off, CHUNK_F)], dstb)

        def step(i, lc):
            vd = dstb[pl.ds(i * 16, 16)]
            vs = srcb[pl.ds(i * 16, 16)]
            dl = vd - lo
            msk = (dl >= 0) & (dl < NPT)
            w = (dl << SRC_BITS) | vs
            mi = jnp.where(msk, 1, 0)
            cum = plsc.cumsum(mi)
            outb[pl.ds(lc, 16)] = dummy
            plsc.store_scatter(outb, [lc + cum - mi], w, mask=msk)
            return lc + cum[15]

        lc = lax.fori_loop(0, CHUNK_F // 16, step, 0)
        outb[pl.ds(lc, 16)] = dummy
        lc = (lc + 7) // 8 * 8
        pltpu.sync_copy(
            outb,
            list_hbm.at[pl.ds(pl.multiple_of(wid * LIST_STRIDE + g, 8),
                              CHUNK_F + 16)])
        return g + lc

    total = lax.fori_loop(0, N_EDGES // CHUNK_F, chunk_body, 0)
    cntb[...] = jnp.full((16,), 1, jnp.int32) * total
    pltpu.sync_copy(cntb,
                    cnt_hbm.at[pl.ds(pl.multiple_of(wid * 16, 8), 16)])


@functools.partial(
    pl.kernel,
    out_type=jax.ShapeDtypeStruct((NPAD, D), jnp.float32),
    mesh=_mesh,
    compiler_params=pltpu.CompilerParams(needs_layout_passes=False),
    scratch_types=[
        pltpu.VMEM((ACC_ROWS, D), jnp.float32),
        pltpu.VMEM((CHUNK_G,), jnp.int32),
        pltpu.VMEM((CHUNK_G,), jnp.int32),
        pltpu.VMEM((CHUNK_G,), jnp.int32),
        pltpu.VMEM((CHUNK_G, D), jnp.float32),
        pltpu.VMEM((16,), jnp.int32),
        pltpu.SemaphoreType.DMA,
    ],
)
def _gather_segmax(m_hbm, list_hbm, cnt_hbm, agg_hbm, acc, idxb, dstb, wb,
                   rows, cntb, sem):
    wid = _wid()
    zero = jnp.zeros((16,), jnp.float32)

    def zrow(i, _):
        for j in range(D // 16):
            acc[i, pl.ds(j * 16, 16)] = zero
        return 0

    lax.fori_loop(0, ACC_ROWS, zrow, 0)

    pltpu.sync_copy(cnt_hbm.at[pl.ds(pl.multiple_of(wid * 16, 8), 16)], cntb)
    k_total = cntb[pl.ds(0, 16)][0]
    nch = (k_total + CHUNK_G - 1) // CHUNK_G

    def chunk(c, _):
        e0 = c * CHUNK_G
        pltpu.sync_copy(
            list_hbm.at[pl.ds(pl.multiple_of(wid * LIST_STRIDE + e0, 8),
                              CHUNK_G)], wb)

        def unpack(i, _):
            w = wb[pl.ds(i * 16, 16)]
            pos = e0 + i * 16 + lax.iota(jnp.int32, 16)
            valid = pos < k_total
            src = w & ((1 << SRC_BITS) - 1)
            dl = lax.shift_right_logical(w, SRC_BITS)
            idxb[pl.ds(i * 16, 16)] = jnp.where(valid, src, 0)
            dstb[pl.ds(i * 16, 16)] = jnp.where(valid, dl, DUMMY_ROW)
            return 0

        lax.fori_loop(0, CHUNK_G // 16, unpack, 0)
        pltpu.async_copy(m_hbm.at[idxb], rows, sem).wait()

        def edge_group(g, _):
            dvec = dstb[pl.ds(g * 16, 16)]
            for lane in range(16):
                dloc = dvec[lane]
                e = g * 16 + lane
                for j in range(D // 16):
                    sl = pl.ds(j * 16, 16)
                    acc[dloc, sl] = jnp.maximum(acc[dloc, sl], rows[e, sl])
            return 0

        lax.fori_loop(0, CHUNK_G // 16, edge_group, 0)
        return 0

    lax.fori_loop(0, nch, chunk, 0)
    pltpu.sync_copy(acc.at[pl.ds(0, NPT)],
                    agg_hbm.at[pl.ds(pl.multiple_of(wid * NPT, 8), NPT)])


_ROWS_BLK = 400
_GRID = N_NODES // _ROWS_BLK


def _dot(a, b):
    return jnp.dot(a, b, preferred_element_type=jnp.float32)


def _tc_stage_a(x_ref, wp, bp, ws, bs, m_ref, s_ref):
    xb = x_ref[...]
    m_ref[...] = jnp.maximum(_dot(xb, wp[...]) + bp[...], 0.0)
    s_ref[...] = _dot(xb, ws[...]) + bs[...]


def _tc_stage_b(s1_ref, agg_ref, wn, bn, wp2, bp2, ws2, bs2, m2_ref, s2_ref):
    h = jnp.maximum(s1_ref[...] + _dot(agg_ref[...], wn[...]) + bn[...], 0.0)
    m2_ref[...] = jnp.maximum(_dot(h, wp2[...]) + bp2[...], 0.0)
    s2_ref[...] = _dot(h, ws2[...]) + bs2[...]


def _tc_stage_c(s2_ref, agg_ref, wn2, bn2, out_ref):
    o = s2_ref[...] + _dot(agg_ref[...], wn2[...]) + bn2[...]
    mx = jnp.max(o, axis=1, keepdims=True)
    e = jnp.exp(o - mx)
    out_ref[...] = (o - mx) - jnp.log(jnp.sum(e, axis=1, keepdims=True))


_row_spec = pl.BlockSpec((_ROWS_BLK, D), lambda i: (i, 0))
_w_spec = pl.BlockSpec((D, D), lambda i: (0, 0))
_b_spec = pl.BlockSpec((1, D), lambda i: (0, 0))
_fdt = jax.ShapeDtypeStruct((N_NODES, D), jnp.float32)


def kernel(x, edge_index, W_pool1, b_pool1, W_self1, b_self1, W_neigh1,
           b_neigh1, W_pool2, b_pool2, W_self2, b_self2, W_neigh2, b_neigh2):
    src = edge_index[0]
    dst = edge_index[1]

    edge_list, edge_cnt = _partition_edges(src, dst)

    m1, s1 = pl.pallas_call(
        _tc_stage_a,
        grid=(_GRID,),
        in_specs=[_row_spec, _w_spec, _b_spec, _w_spec, _b_spec],
        out_specs=[_row_spec, _row_spec],
        out_shape=[_fdt, _fdt],
    )(x, W_pool1, b_pool1.reshape(1, D), W_self1, b_self1.reshape(1, D))

    agg1 = _gather_segmax(m1, edge_list, edge_cnt)[:N_NODES]

    m2, s2 = pl.pallas_call(
        _tc_stage_b,
        grid=(_GRID,),
        in_specs=[_row_spec, _row_spec, _w_spec, _b_spec, _w_spec, _b_spec,
                  _w_spec, _b_spec],
        out_specs=[_row_spec, _row_spec],
        out_shape=[_fdt, _fdt],
    )(s1, agg1, W_neigh1, b_neigh1.reshape(1, D), W_pool2,
      b_pool2.reshape(1, D), W_self2, b_self2.reshape(1, D))

    agg2 = _gather_segmax(m2, edge_list, edge_cnt)[:N_NODES]

    out = pl.pallas_call(
        _tc_stage_c,
        grid=(_GRID,),
        in_specs=[_row_spec, _row_spec, _w_spec, _b_spec],
        out_specs=_row_spec,
        out_shape=_fdt,
    )(s2, agg2, W_neigh2, b_neigh2.reshape(1, D))

    return out
